# baseline (speedup 1.0000x reference)
import jax
import jax.numpy as jnp
from jax.experimental import pallas as pl
from jax.experimental.pallas import tpu as pltpu

_B = 8
_N = 2048
_TQ = 256


def _extract_min(d, iota, n):
    m = jnp.min(d, axis=1, keepdims=True)
    am = jnp.min(jnp.where(d == m, iota, n), axis=1, keepdims=True)
    return iota == am


def _gather_table(oh, table):
    return jax.lax.dot_general(
        oh.astype(jnp.float32), table, (((1,), (0,)), ((), ())),
        precision=jax.lax.Precision.HIGHEST,
        preferred_element_type=jnp.float32)


def _dist(q, pts):
    cross = jax.lax.dot_general(
        q.astype(jnp.bfloat16), pts.astype(jnp.bfloat16),
        (((1,), (0,)), ((), ())), preferred_element_type=jnp.float32)
    qn = jnp.sum(q * q, axis=1, keepdims=True)
    pn = jnp.sum(pts * pts, axis=0, keepdims=True)
    return (qn + pn) - 2.0 * cross


def _top3_kappa(d, ptsT, qx, qy, qz, nx, ny, nz):
    tq, n = d.shape
    iota = jax.lax.broadcasted_iota(jnp.int32, (tq, n), 1)
    oh = _extract_min(d, iota, n)
    d = jnp.where(oh, jnp.inf, d)
    acc = jnp.zeros((tq, 1), jnp.float32)
    for _ in range(2):
        oh = _extract_min(d, iota, n)
        sel = _gather_table(oh, ptsT)
        vx = sel[:, 0:1] - qx
        vy = sel[:, 1:2] - qy
        vz = sel[:, 2:3] - qz
        nrm = jnp.maximum(jnp.sqrt(vx * vx + vy * vy + vz * vz), 1e-12)
        acc += jnp.abs((vx * nx + vy * ny + vz * nz) / nrm)
        d = jnp.where(oh, jnp.inf, d)
    return acc * 0.5


def _kappa_kernel(ptsT_ref, pts_ref, ptsTf_ref, nrmT_ref, kap_ref):
    q = ptsT_ref[0]
    pts = pts_ref[0]
    ptsT = ptsTf_ref[0]
    nq = nrmT_ref[0]
    d = _dist(q, pts)
    kap = _top3_kappa(d, ptsT, q[:, 0:1], q[:, 1:2], q[:, 2:3],
                      nq[:, 0:1], nq[:, 1:2], nq[:, 2:3])
    kap_ref[0] = kap


def _adv_kernel(advT_ref, adv_ref, advTf_ref, ori_ref, tbl_ref, out_ref):
    q = advT_ref[0]
    adv = adv_ref[0]
    advT = advTf_ref[0]
    ori = ori_ref[0]
    tbl = tbl_ref[0]
    qx, qy, qz = q[:, 0:1], q[:, 1:2], q[:, 2:3]

    d1 = _dist(q, ori)
    tq, n = d1.shape
    iota = jax.lax.broadcasted_iota(jnp.int32, (tq, n), 1)
    oh1 = _extract_min(d1, iota, n)
    sel = _gather_table(oh1, tbl)
    nx, ny, nz, ok = sel[:, 0:1], sel[:, 1:2], sel[:, 2:3], sel[:, 3:4]

    d2 = _dist(q, adv)
    kap = _top3_kappa(d2, advT, qx, qy, qz, nx, ny, nz)

    diff = kap - ok
    part = jnp.sum(diff * diff, axis=0, keepdims=True)

    @pl.when(pl.program_id(1) == 0)
    def _():
        out_ref[0] = jnp.zeros((1, 1), jnp.float32)

    out_ref[0] += part


def kernel(ori_data, adv_data, ori_normal):
    b, _, n = ori_data.shape
    oriT = jnp.transpose(ori_data, (0, 2, 1))
    advT = jnp.transpose(adv_data, (0, 2, 1))
    onrmT = jnp.transpose(ori_normal, (0, 2, 1))

    grid = (b, n // _TQ)
    ori_kappa = pl.pallas_call(
        _kappa_kernel,
        grid=grid,
        in_specs=[
            pl.BlockSpec((1, _TQ, 3), lambda i, j: (i, j, 0)),
            pl.BlockSpec((1, 3, n), lambda i, j: (i, 0, 0)),
            pl.BlockSpec((1, n, 3), lambda i, j: (i, 0, 0)),
            pl.BlockSpec((1, _TQ, 3), lambda i, j: (i, j, 0)),
        ],
        out_specs=pl.BlockSpec((1, _TQ, 1), lambda i, j: (i, j, 0)),
        out_shape=jax.ShapeDtypeStruct((b, n, 1), jnp.float32),
        compiler_params=pltpu.CompilerParams(
            dimension_semantics=("parallel", "parallel")),
    )(oriT, ori_data, oriT, onrmT)

    tbl = jnp.concatenate([onrmT, ori_kappa], axis=-1)

    partials = pl.pallas_call(
        _adv_kernel,
        grid=grid,
        in_specs=[
            pl.BlockSpec((1, _TQ, 3), lambda i, j: (i, j, 0)),
            pl.BlockSpec((1, 3, n), lambda i, j: (i, 0, 0)),
            pl.BlockSpec((1, n, 3), lambda i, j: (i, 0, 0)),
            pl.BlockSpec((1, 3, n), lambda i, j: (i, 0, 0)),
            pl.BlockSpec((1, n, 4), lambda i, j: (i, 0, 0)),
        ],
        out_specs=pl.BlockSpec((1, 1, 1), lambda i, j: (i, 0, 0)),
        out_shape=jax.ShapeDtypeStruct((b, 1, 1), jnp.float32),
        compiler_params=pltpu.CompilerParams(
            dimension_semantics=("parallel", "arbitrary")),
    )(advT, adv_data, advT, ori_data, tbl)

    return jnp.sum(partials) / (b * n)


# speedup vs baseline: 1.5599x; 1.5599x over previous
import jax
import jax.numpy as jnp
from jax.experimental import pallas as pl
from jax.experimental.pallas import tpu as pltpu

_B = 8
_N = 2048
_TQ = 512


def _extract_min(d, iota, n):
    m = jnp.min(d, axis=1, keepdims=True)
    am = jnp.min(jnp.where(d == m, iota, n), axis=1, keepdims=True)
    return iota == am


def _sel(oh, row):
    return jnp.sum(jnp.where(oh, row, 0.0), axis=1, keepdims=True)


def _r16(v):
    return v.astype(jnp.bfloat16).astype(jnp.float32)


def _dist(qx, qy, qz, px, py, pz):
    qn = qx * qx + qy * qy + qz * qz
    pn = px * px + py * py + pz * pz
    cross = _r16(qx) * _r16(px) + _r16(qy) * _r16(py) + _r16(qz) * _r16(pz)
    return (qn + pn) - 2.0 * cross


def _top3_kappa(d, px, py, pz, qx, qy, qz, nx, ny, nz):
    tq, n = d.shape
    iota = jax.lax.broadcasted_iota(jnp.int32, (tq, n), 1)
    oh = _extract_min(d, iota, n)
    d = jnp.where(oh, jnp.inf, d)
    acc = jnp.zeros((tq, 1), jnp.float32)
    for _ in range(2):
        oh = _extract_min(d, iota, n)
        vx = _sel(oh, px) - qx
        vy = _sel(oh, py) - qy
        vz = _sel(oh, pz) - qz
        nrm = jnp.maximum(jnp.sqrt(vx * vx + vy * vy + vz * vz), 1e-12)
        acc += jnp.abs((vx * nx + vy * ny + vz * nz) / nrm)
        d = jnp.where(oh, jnp.inf, d)
    return acc * 0.5


def _kappa_kernel(ptsT_ref, pts_ref, nrmT_ref, kap_ref):
    q = ptsT_ref[0]
    pts = pts_ref[0]
    nq = nrmT_ref[0]
    px, py, pz = pts[0:1, :], pts[1:2, :], pts[2:3, :]
    qx, qy, qz = q[:, 0:1], q[:, 1:2], q[:, 2:3]
    d = _dist(qx, qy, qz, px, py, pz)
    kap = _top3_kappa(d, px, py, pz, qx, qy, qz,
                      nq[:, 0:1], nq[:, 1:2], nq[:, 2:3])
    kap_ref[0] = kap


def _adv_kernel(advT_ref, adv_ref, ori_ref, onrm_ref, okap_ref, out_ref):
    q = advT_ref[0]
    adv = adv_ref[0]
    ori = ori_ref[0]
    onrm = onrm_ref[0]
    okap = okap_ref[0]
    qx, qy, qz = q[:, 0:1], q[:, 1:2], q[:, 2:3]

    d1 = _dist(qx, qy, qz, ori[0:1, :], ori[1:2, :], ori[2:3, :])
    tq, n = d1.shape
    iota = jax.lax.broadcasted_iota(jnp.int32, (tq, n), 1)
    oh1 = _extract_min(d1, iota, n)
    nx = _sel(oh1, onrm[0:1, :])
    ny = _sel(oh1, onrm[1:2, :])
    nz = _sel(oh1, onrm[2:3, :])
    ok = _sel(oh1, okap)

    px, py, pz = adv[0:1, :], adv[1:2, :], adv[2:3, :]
    d2 = _dist(qx, qy, qz, px, py, pz)
    kap = _top3_kappa(d2, px, py, pz, qx, qy, qz, nx, ny, nz)

    diff = kap - ok
    part = jnp.sum(diff * diff, axis=0, keepdims=True)

    @pl.when(pl.program_id(1) == 0)
    def _():
        out_ref[0] = jnp.zeros((1, 1), jnp.float32)

    out_ref[0] += part


def kernel(ori_data, adv_data, ori_normal):
    b, _, n = ori_data.shape
    oriT = jnp.transpose(ori_data, (0, 2, 1))
    advT = jnp.transpose(adv_data, (0, 2, 1))
    onrmT = jnp.transpose(ori_normal, (0, 2, 1))

    grid = (b, n // _TQ)
    ori_kappa = pl.pallas_call(
        _kappa_kernel,
        grid=grid,
        in_specs=[
            pl.BlockSpec((1, _TQ, 3), lambda i, j: (i, j, 0)),
            pl.BlockSpec((1, 3, n), lambda i, j: (i, 0, 0)),
            pl.BlockSpec((1, _TQ, 3), lambda i, j: (i, j, 0)),
        ],
        out_specs=pl.BlockSpec((1, _TQ, 1), lambda i, j: (i, j, 0)),
        out_shape=jax.ShapeDtypeStruct((b, n, 1), jnp.float32),
        compiler_params=pltpu.CompilerParams(
            dimension_semantics=("parallel", "parallel")),
    )(oriT, ori_data, onrmT)

    okap_row = jnp.transpose(ori_kappa, (0, 2, 1))

    partials = pl.pallas_call(
        _adv_kernel,
        grid=grid,
        in_specs=[
            pl.BlockSpec((1, _TQ, 3), lambda i, j: (i, j, 0)),
            pl.BlockSpec((1, 3, n), lambda i, j: (i, 0, 0)),
            pl.BlockSpec((1, 3, n), lambda i, j: (i, 0, 0)),
            pl.BlockSpec((1, 3, n), lambda i, j: (i, 0, 0)),
            pl.BlockSpec((1, 1, n), lambda i, j: (i, 0, 0)),
        ],
        out_specs=pl.BlockSpec((1, 1, 1), lambda i, j: (i, 0, 0)),
        out_shape=jax.ShapeDtypeStruct((b, 1, 1), jnp.float32),
        compiler_params=pltpu.CompilerParams(
            dimension_semantics=("parallel", "arbitrary")),
    )(advT, adv_data, ori_data, ori_normal, okap_row)

    return jnp.sum(partials) / (b * n)
